# baseline (speedup 1.0000x reference)
import functools
import jax
import jax.numpy as jnp
from jax import lax
from jax.experimental import pallas as pl
from jax.experimental.pallas import tpu as pltpu
from jax.experimental.pallas import tpu_sc as plsc

_POOLED_H = 7
_POOLED_W = 7
_SPATIAL_SCALE = 0.25
_SAMPLING_RATIO = 2
_GAMMA = 0.1

_L = 16
_NW = 32
_NBINS_PER_ROI = _POOLED_H * _POOLED_W


def _sc_deform_pool(table, rois_t, offw, offh, *, n_rois, nb_pad, h, w, c):
    bins_per_w = nb_pad // _NW
    n_groups = bins_per_w // _L
    hw = h * w

    mesh = plsc.VectorSubcoreMesh(core_axis_name="c", subcore_axis_name="s")

    @functools.partial(
        pl.kernel,
        out_type=jax.ShapeDtypeStruct((nb_pad, c), jnp.float32),
        mesh=mesh,
        compiler_params=pltpu.CompilerParams(
            needs_layout_passes=False, use_tc_tiling_on_sc=False),
        scratch_types=[
            pltpu.VMEM((5 * n_rois,), jnp.float32),
            pltpu.VMEM((bins_per_w,), jnp.float32),
            pltpu.VMEM((bins_per_w,), jnp.float32),
            pltpu.VMEM((2, 128), jnp.int32),
            pltpu.VMEM((256, c), jnp.float32),
            pltpu.VMEM((256,), jnp.float32),
            pltpu.VMEM((_L, c), jnp.float32),
            pltpu.SemaphoreType.DMA,
        ],
    )
    def body(table_h, rois_h, offw_h, offh_h, out_h, rois_v, offw_v, offh_v,
             idx_v, rows_v, wts_v, outb_v, sem):
        wid = lax.axis_index("s") * 2 + lax.axis_index("c")
        base_bin = wid * bins_per_w

        pltpu.sync_copy(rois_h, rois_v)
        pltpu.sync_copy(offw_h.at[pl.ds(base_bin, bins_per_w)], offw_v)
        pltpu.sync_copy(offh_h.at[pl.ds(base_bin, bins_per_w)], offh_v)

        lane = lax.iota(jnp.int32, _L)
        lane_f = lane.astype(jnp.float32)

        def group_body(g, carry):
            b_i = base_bin + g * _L + lane
            b_f = b_i.astype(jnp.float32)
            n_i = ((b_f + 0.5) * (1.0 / _NBINS_PER_ROI)).astype(jnp.int32)
            n_i = jnp.minimum(n_i, n_rois - 1)
            bin_i = b_i - _NBINS_PER_ROI * n_i
            bin_f = bin_i.astype(jnp.float32)
            phh = ((bin_f + 0.5) * (1.0 / _POOLED_W)).astype(jnp.int32)
            phh_f = phh.astype(jnp.float32)
            pww_f = bin_f - _POOLED_W * phh_f

            bat = plsc.load_gather(rois_v, [n_i])
            x1 = plsc.load_gather(rois_v, [n_i + n_rois])
            y1 = plsc.load_gather(rois_v, [n_i + 2 * n_rois])
            x2 = plsc.load_gather(rois_v, [n_i + 3 * n_rois])
            y2 = plsc.load_gather(rois_v, [n_i + 4 * n_rois])

            roi_sw = x1 * _SPATIAL_SCALE - 0.5
            roi_sh = y1 * _SPATIAL_SCALE - 0.5
            roi_w = (x2 * _SPATIAL_SCALE - 0.5) - roi_sw
            roi_h = (y2 * _SPATIAL_SCALE - 0.5) - roi_sh
            bin_w = roi_w * (1.0 / _POOLED_W)
            bin_h = roi_h * (1.0 / _POOLED_H)

            ow = offw_v[pl.ds(g * _L, _L)]
            oh = offh_v[pl.ds(g * _L, _L)]
            start_w = roi_sw + _GAMMA * roi_w * ow
            start_h = roi_sh + _GAMMA * roi_h * oh

            base_i = bat.astype(jnp.int32) * hw

            for s in range(_SAMPLING_RATIO * _SAMPLING_RATIO):
                iy = s // _SAMPLING_RATIO
                ix = s % _SAMPLING_RATIO
                cy = (iy + 0.5) / _SAMPLING_RATIO
                cx = (ix + 0.5) / _SAMPLING_RATIO
                y = (start_h + phh_f * bin_h) + cy * bin_h
                x = (start_w + pww_f * bin_w) + cx * bin_w
                valid = ((y >= -1.0) & (y <= float(h))
                         & (x >= -1.0) & (x <= float(w)))
                yc = jnp.minimum(jnp.maximum(y, 0.0), float(h))
                xc = jnp.minimum(jnp.maximum(x, 0.0), float(w))
                ylf = yc.astype(jnp.int32).astype(jnp.float32)
                xlf = xc.astype(jnp.int32).astype(jnp.float32)
                ycond = ylf >= float(h - 1)
                xcond = xlf >= float(w - 1)
                ylf = jnp.where(ycond, float(h - 1), ylf)
                xlf = jnp.where(xcond, float(w - 1), xlf)
                yhf = jnp.where(ycond, float(h - 1), ylf + 1.0)
                xhf = jnp.where(xcond, float(w - 1), xlf + 1.0)
                yc = jnp.where(ycond, ylf, yc)
                xc = jnp.where(xcond, xlf, xc)
                ly = yc - ylf
                lx = xc - xlf
                hy = 1.0 - ly
                hx = 1.0 - lx
                vf = jnp.where(valid,
                               1.0 / (_SAMPLING_RATIO * _SAMPLING_RATIO), 0.0)
                y_lo = ylf.astype(jnp.int32)
                y_hi = yhf.astype(jnp.int32)
                x_lo = xlf.astype(jnp.int32)
                x_hi = xhf.astype(jnp.int32)
                r_ll = base_i + y_lo * w + x_lo
                r_lh = base_i + y_lo * w + x_hi
                r_hl = base_i + y_hi * w + x_lo
                r_hh = base_i + y_hi * w + x_hi
                ws = (hy * hx * vf, hy * lx * vf, ly * hx * vf, ly * lx * vf)
                rs = (r_ll, r_lh, r_hl, r_hh)
                for corner in range(4):
                    k = s * 4 + corner
                    idx_v[k // 8, pl.ds((k % 8) * _L, _L)] = rs[corner]
                    wts_v[pl.ds(k * _L, _L)] = ws[corner]

            cp0 = pltpu.async_copy(table_h.at[idx_v.at[0]],
                                   rows_v.at[pl.ds(0, 128)], sem)
            cp1 = pltpu.async_copy(table_h.at[idx_v.at[1]],
                                   rows_v.at[pl.ds(128, 128)], sem)
            cp0.wait()
            cp1.wait()

            for chunk in range(c // _L):
                c0 = chunk * _L

                def acc_body(k, acc):
                    wk = wts_v[pl.ds(k * _L, _L)]
                    row0 = k * _L + lane
                    new = []
                    for j in range(_L):
                        col = jnp.full((_L,), c0 + j, jnp.int32)
                        v = plsc.load_gather(rows_v, [row0, col])
                        new.append(acc[j] + wk * v)
                    return tuple(new)

                acc0 = tuple(jnp.zeros((_L,), jnp.float32) for _ in range(_L))
                acc = lax.fori_loop(0, 16, acc_body, acc0)
                for j in range(_L):
                    col = jnp.full((_L,), c0 + j, jnp.int32)
                    plsc.store_scatter(outb_v, [lane, col], acc[j])

            pltpu.sync_copy(outb_v, out_h.at[pl.ds(base_bin + g * _L, _L)])
            return carry

        lax.fori_loop(0, n_groups, group_body, 0)

    return body(table, rois_t, offw, offh)


def kernel(input, rois, offset):
    b, c, h, w = input.shape
    n = rois.shape[0]
    nb = n * _NBINS_PER_ROI
    nb_pad = ((nb + _NW * _L - 1) // (_NW * _L)) * (_NW * _L)

    table = jnp.transpose(input, (0, 2, 3, 1)).reshape(b * h * w, c)
    rois_t = jnp.transpose(rois, (1, 0)).reshape(-1)
    off = jnp.transpose(offset.reshape(n, 2, nb // n), (1, 0, 2)).reshape(2, nb)
    off = jnp.pad(off, ((0, 0), (0, nb_pad - nb)))

    out = _sc_deform_pool(table, rois_t, off[0], off[1],
                          n_rois=n, nb_pad=nb_pad, h=h, w=w, c=c)
    out = out[:nb].reshape(n, _POOLED_H, _POOLED_W, c)
    return jnp.transpose(out, (0, 3, 1, 2))
